# baseline (speedup 1.0000x reference)
import functools

import jax
import jax.numpy as jnp
from jax import lax
from jax.experimental import pallas as pl
from jax.experimental.pallas import tpu as pltpu
from jax.experimental.pallas import tpu_sc as plsc

_BLK = 128
_NCLS = 3


def _sc_body(n_rows, rows_per_w, flat_pre, labels, score_out, pred_out,
             laboh_out, idxa_v, idxb_v, idxc_v, oi0_v, oi1_v, oi2_v, ga_v,
             gb_v, gc_v, lab_v, score_v, pred_v, laboh_v, sem):
    nc = 2
    wid = lax.axis_index("s") * nc + lax.axis_index("c")
    iota = lax.iota(jnp.int32, 16)
    n_blk = rows_per_w // _BLK
    d = 1000
    for b in range(n_blk):
        base = wid * rows_per_w + b * _BLK
        for s in range(_BLK // 16):
            sl = pl.ds(s * 16, 16)
            row = base + s * 16 + iota
            t = row * d
            idxa_v[sl] = t
            idxb_v[sl] = t + 1
            idxc_v[sl] = t + 2
            q = row * _NCLS
            oi0_v[sl] = q
            oi1_v[sl] = q + 1
            oi2_v[sl] = q + 2
        cps = [
            pltpu.async_copy(flat_pre.at[idxa_v], ga_v, sem),
            pltpu.async_copy(flat_pre.at[idxb_v], gb_v, sem),
            pltpu.async_copy(flat_pre.at[idxc_v], gc_v, sem),
            pltpu.async_copy(labels.at[pl.ds(base, _BLK)], lab_v, sem),
        ]
        for cp in cps:
            cp.wait()
        for c in range(_BLK // 16):
            sl = pl.ds(c * 16, 16)
            v0 = ga_v[sl]
            v1 = gb_v[sl]
            v2 = gc_v[sl]
            m = jnp.maximum(v0, jnp.maximum(v1, v2))
            e0 = jnp.exp(v0 - m)
            e1 = jnp.exp(v1 - m)
            e2 = jnp.exp(v2 - m)
            inv = 1.0 / (e0 + e1 + e2)
            one = jnp.full((16,), 1.0, jnp.float32)
            zero = jnp.full((16,), 0.0, jnp.float32)
            f0 = (jnp.where(v0 >= v1, one, zero)
                  * jnp.where(v0 >= v2, one, zero))
            f1 = (one - f0) * jnp.where(v1 >= v2, one, zero)
            f2 = one - f0 - f1
            lab = lab_v[sl]
            for j, (yj, pj) in enumerate(((e0, f0), (e1, f1), (e2, f2))):
                osl = pl.ds(j * _BLK + c * 16, 16)
                score_v[osl] = yj * inv
                pred_v[osl] = pj
                laboh_v[osl] = jnp.where(lab == j, one, zero)
        cps = []
        for j, oidx in enumerate((oi0_v, oi1_v, oi2_v)):
            jsl = pl.ds(j * _BLK, _BLK)
            cps.append(pltpu.async_copy(score_v.at[jsl],
                                        score_out.at[oidx], sem))
            cps.append(pltpu.async_copy(pred_v.at[jsl],
                                        pred_out.at[oidx], sem))
            cps.append(pltpu.async_copy(laboh_v.at[jsl],
                                        laboh_out.at[oidx], sem))
        for cp in cps:
            cp.wait()


def kernel(pre, y_label, stage_name):
    n, d = pre.shape
    flat_pre = pre.reshape(n * d)
    labels = y_label.astype(jnp.int32)

    info = plsc.get_sparse_core_info()
    n_workers = info.num_cores * info.num_subcores
    rows_per_w = n // n_workers
    mesh = plsc.VectorSubcoreMesh(core_axis_name="c", subcore_axis_name="s")

    flat = jax.ShapeDtypeStruct((n * _NCLS,), jnp.float32)
    fvec = pltpu.VMEM((_BLK,), jnp.float32)
    ivec = pltpu.VMEM((_BLK,), jnp.int32)
    obuf = pltpu.VMEM((_BLK * _NCLS,), jnp.float32)
    k = functools.partial(
        pl.kernel,
        out_type=(flat, flat, flat),
        mesh=mesh,
        scratch_types=[
            ivec, ivec, ivec, ivec, ivec, ivec,
            fvec, fvec, fvec, ivec,
            obuf, obuf, obuf,
            pltpu.SemaphoreType.DMA,
        ],
    )(functools.partial(_sc_body, n, rows_per_w))
    score, pred_oh, lab_oh = k(flat_pre, labels)
    shape3 = (n, _NCLS)
    return (score.reshape(shape3), pred_oh.reshape(shape3),
            lab_oh.reshape(shape3))


# speedup vs baseline: 2.5644x; 2.5644x over previous
import functools

import jax
import jax.numpy as jnp
from jax import lax
from jax.experimental import pallas as pl
from jax.experimental.pallas import tpu as pltpu
from jax.experimental.pallas import tpu_sc as plsc

_CHUNK = 384
_NCLS = 3


def _sc_body(n_rows, lanes_per_w, flat_pre, labels, score_out, pred_out,
             laboh_out, idxa_v, idxb_v, idxc_v, idxl_v, ga_v, gb_v, gc_v,
             glab_v, score_v, pred_v, laboh_v, sem):
    nc = 2
    wid = lax.axis_index("s") * nc + lax.axis_index("c")
    iota = lax.iota(jnp.int32, 16)
    three = jnp.full((16,), 3, jnp.int32)
    d = 1000
    n_chunk = lanes_per_w // _CHUNK
    for t in range(n_chunk):
        gl = wid * lanes_per_w + t * _CHUNK
        for s in range(_CHUNK // 16):
            sl = pl.ds(s * 16, 16)
            rows = lax.div(gl + s * 16 + iota, three)
            fa = rows * d
            idxa_v[sl] = fa
            idxb_v[sl] = fa + 1
            idxc_v[sl] = fa + 2
            idxl_v[sl] = rows
        cps = []
        for k in range(_CHUNK // 128):
            ksl = pl.ds(k * 128, 128)
            for isrc, dst in ((idxa_v, ga_v), (idxb_v, gb_v),
                             (idxc_v, gc_v)):
                cps.append(pltpu.async_copy(flat_pre.at[isrc.at[ksl]],
                                            dst.at[ksl], sem))
            cps.append(pltpu.async_copy(labels.at[idxl_v.at[ksl]],
                                        glab_v.at[ksl], sem))
        for cp in cps:
            cp.wait()
        one = jnp.full((16,), 1.0, jnp.float32)
        zero = jnp.full((16,), 0.0, jnp.float32)
        for s in range(_CHUNK // 16):
            sl = pl.ds(s * 16, 16)
            v0 = ga_v[sl]
            v1 = gb_v[sl]
            v2 = gc_v[sl]
            m = jnp.maximum(v0, jnp.maximum(v1, v2))
            e0 = jnp.exp(v0 - m)
            e1 = jnp.exp(v1 - m)
            e2 = jnp.exp(v2 - m)
            inv = 1.0 / (e0 + e1 + e2)
            cls = lax.rem(jnp.full((16,), (16 * s) % 3, jnp.int32) + iota,
                          three)
            is1 = cls == 1
            score_v[sl] = jnp.where(
                cls == 0, e0, jnp.where(is1, e1, e2)) * inv
            f0 = (jnp.where(v0 >= v1, one, zero)
                  * jnp.where(v0 >= v2, one, zero))
            f1 = (one - f0) * jnp.where(v1 >= v2, one, zero)
            f2 = one - f0 - f1
            pred_v[sl] = jnp.where(cls == 0, f0, jnp.where(is1, f1, f2))
            laboh_v[sl] = jnp.where(glab_v[sl] == cls, one, zero)
        cps = [
            pltpu.async_copy(score_v, score_out.at[pl.ds(gl, _CHUNK)], sem),
            pltpu.async_copy(pred_v, pred_out.at[pl.ds(gl, _CHUNK)], sem),
            pltpu.async_copy(laboh_v, laboh_out.at[pl.ds(gl, _CHUNK)], sem),
        ]
        for cp in cps:
            cp.wait()


def kernel(pre, y_label, stage_name):
    n, d = pre.shape
    flat_pre = pre.reshape(n * d)
    labels = y_label.astype(jnp.int32)

    info = plsc.get_sparse_core_info()
    n_workers = info.num_cores * info.num_subcores
    lanes_per_w = n * _NCLS // n_workers
    mesh = plsc.VectorSubcoreMesh(core_axis_name="c", subcore_axis_name="s")

    flat = jax.ShapeDtypeStruct((n * _NCLS,), jnp.float32)
    fvec = pltpu.VMEM((_CHUNK,), jnp.float32)
    ivec = pltpu.VMEM((_CHUNK,), jnp.int32)
    k = functools.partial(
        pl.kernel,
        out_type=(flat, flat, flat),
        mesh=mesh,
        scratch_types=[
            ivec, ivec, ivec, ivec,
            fvec, fvec, fvec, ivec,
            fvec, fvec, fvec,
            pltpu.SemaphoreType.DMA,
        ],
    )(functools.partial(_sc_body, n, lanes_per_w))
    score, pred_oh, lab_oh = k(flat_pre, labels)
    shape3 = (n, _NCLS)
    return (score.reshape(shape3), pred_oh.reshape(shape3),
            lab_oh.reshape(shape3))


# speedup vs baseline: 4.1911x; 1.6344x over previous
import functools

import jax
import jax.numpy as jnp
from jax import lax
from jax.experimental import pallas as pl
from jax.experimental.pallas import tpu as pltpu
from jax.experimental.pallas import tpu_sc as plsc

_BLK = 128
_NCLS = 3


def _sc_body(n_rows, rows_per_w, pre_hbm, labels, score_out, pred_out,
             laboh_out, buf2d, lab_v, score_v, pred_v, laboh_v, sem):
    nc = 2
    wid = lax.axis_index("s") * nc + lax.axis_index("c")
    n_blk = rows_per_w // _BLK
    one = jnp.full((16,), 1.0, jnp.float32)
    zero = jnp.full((16,), 0.0, jnp.float32)
    iota = lax.iota(jnp.int32, 16)
    for b in range(n_blk):
        base = wid * rows_per_w + b * _BLK
        rsl = pl.ds(base, _BLK)
        cps = [
            pltpu.async_copy(pre_hbm.at[rsl, pl.ds(0, 128)], buf2d, sem),
            pltpu.async_copy(labels.at[rsl], lab_v, sem),
        ]
        for cp in cps:
            cp.wait()
        for c in range(_BLK // 16):
            sl = pl.ds(c * 16, 16)
            row = c * 16 + iota
            v0 = plsc.load_gather(buf2d, [row, jnp.full((16,), 0,
                                                        jnp.int32)])
            v1 = plsc.load_gather(buf2d, [row, jnp.full((16,), 1,
                                                        jnp.int32)])
            v2 = plsc.load_gather(buf2d, [row, jnp.full((16,), 2,
                                                        jnp.int32)])
            m = jnp.maximum(v0, jnp.maximum(v1, v2))
            e0 = jnp.exp(v0 - m)
            e1 = jnp.exp(v1 - m)
            e2 = jnp.exp(v2 - m)
            inv = 1.0 / (e0 + e1 + e2)
            f0 = (jnp.where(v0 >= v1, one, zero)
                  * jnp.where(v0 >= v2, one, zero))
            f1 = (one - f0) * jnp.where(v1 >= v2, one, zero)
            f2 = one - f0 - f1
            lab = lab_v[sl]
            tri = row * _NCLS
            for j, (yj, pj) in enumerate(((e0, f0), (e1, f1), (e2, f2))):
                plsc.store_scatter(score_v, [tri + j], yj * inv)
                plsc.store_scatter(pred_v, [tri + j], pj)
                plsc.store_scatter(laboh_v, [tri + j],
                                   jnp.where(lab == j, one, zero))
        obase = base * _NCLS
        osz = _BLK * _NCLS
        cps = [
            pltpu.async_copy(score_v, score_out.at[pl.ds(obase, osz)], sem),
            pltpu.async_copy(pred_v, pred_out.at[pl.ds(obase, osz)], sem),
            pltpu.async_copy(laboh_v, laboh_out.at[pl.ds(obase, osz)], sem),
        ]
        for cp in cps:
            cp.wait()


def kernel(pre, y_label, stage_name):
    n, d = pre.shape
    labels = y_label.astype(jnp.int32)

    info = plsc.get_sparse_core_info()
    n_workers = info.num_cores * info.num_subcores
    rows_per_w = n // n_workers
    mesh = plsc.VectorSubcoreMesh(core_axis_name="c", subcore_axis_name="s")

    flat = jax.ShapeDtypeStruct((n * _NCLS,), jnp.float32)
    obuf = pltpu.VMEM((_BLK * _NCLS,), jnp.float32)
    k = functools.partial(
        pl.kernel,
        out_type=(flat, flat, flat),
        mesh=mesh,
        compiler_params=pltpu.CompilerParams(needs_layout_passes=False),
        scratch_types=[
            pltpu.VMEM((_BLK, 128), jnp.float32),
            pltpu.VMEM((_BLK,), jnp.int32),
            obuf, obuf, obuf,
            pltpu.SemaphoreType.DMA,
        ],
    )(functools.partial(_sc_body, n, rows_per_w))
    score, pred_oh, lab_oh = k(pre, labels)
    shape3 = (n, _NCLS)
    return (score.reshape(shape3), pred_oh.reshape(shape3),
            lab_oh.reshape(shape3))


# speedup vs baseline: 5.1839x; 1.2369x over previous
import jax
import jax.numpy as jnp
from jax import lax
from jax.experimental import pallas as pl


_ROWS_PER_BLK = 1024
_NC = 3


def _body(pre_ref, lab_ref, score_ref, pred_oh_ref, lab_oh_ref):
    x = pre_ref[...]
    lane = lax.broadcasted_iota(jnp.int32, x.shape, 1)
    valid = lane < _NC
    neg_inf = jnp.float32(-jnp.inf)
    xm = jnp.where(valid, x, neg_inf)
    m = jnp.max(xm, axis=1, keepdims=True)
    e = jnp.where(valid, jnp.exp(x - m), 0.0)
    s = jnp.sum(e, axis=1, keepdims=True)
    y = e / s

    big = jnp.int32(10**6)
    idx = jnp.where(valid & (xm == m), lane, big)
    pred = jnp.min(idx, axis=1, keepdims=True)

    score_ref[...] = y
    pred_oh_ref[...] = ((lane == pred) & valid).astype(jnp.float32)
    lab = lab_ref[...]
    lab_oh_ref[...] = ((lane == lab) & valid).astype(jnp.float32)


def kernel(pre, y_label, stage_name):
    n, _ = pre.shape
    grid = n // _ROWS_PER_BLK
    lab2d = y_label.reshape(n, 1).astype(jnp.int32)
    out_shapes = (
        jax.ShapeDtypeStruct((n, 128), jnp.float32),
        jax.ShapeDtypeStruct((n, 128), jnp.float32),
        jax.ShapeDtypeStruct((n, 128), jnp.float32),
    )
    o_spec = pl.BlockSpec((_ROWS_PER_BLK, 128), lambda i: (i, 0))
    score, pred_oh, lab_oh = pl.pallas_call(
        _body,
        grid=(grid,),
        in_specs=[
            pl.BlockSpec((_ROWS_PER_BLK, 128), lambda i: (i, 0)),
            pl.BlockSpec((_ROWS_PER_BLK, 1), lambda i: (i, 0)),
        ],
        out_specs=(o_spec, o_spec, o_spec),
        out_shape=out_shapes,
    )(pre, lab2d)
    return (score[:, :_NC], pred_oh[:, :_NC], lab_oh[:, :_NC])
